# Initial kernel scaffold; baseline (speedup 1.0000x reference)
import jax
import jax.numpy as jnp
from jax.experimental import pallas as pl


def kernel(x, k_star, w_qw, w_qr, w_v, w_o):
    raise NotImplementedError("write your pallas kernel here")


# speedup vs baseline: 21.1038x; 21.1038x over previous
import math

import jax
import jax.numpy as jnp
from jax.experimental import pallas as pl
from jax.experimental.pallas import tpu as pltpu

_N, _T, _C = 4, 4096, 1024
_H, _DQK, _DV, _L = 8, 64, 64, 64
_S = 256
_NB = _T // _S


def _softmax_cols(s):
    m = jnp.max(s, axis=0, keepdims=True)
    e = jnp.exp(s - m)
    return e / jnp.sum(e, axis=0, keepdims=True)


def _shear_t(p, masks):
    out = p
    for b in range(6):
        rolled = jnp.roll(out, -(1 << b), axis=0)
        out = jnp.where(masks[b], rolled, out)
    return out


def _body(x_ref, wqw_ref, wqr_ref, wv_ref, ks_ref, wo_ref, o_ref, y_scr, v_scr):
    b = pl.program_id(1)

    @pl.when(b == 0)
    def _():
        y_scr[...] = jnp.zeros_like(y_scr)

    ks = ks_ref[...]
    scale = 1.0 / math.sqrt(_DQK)

    xt = x_ref[0].T
    qwt = jnp.dot(wqw_ref[...], xt, preferred_element_type=jnp.float32)
    vt = jnp.dot(wv_ref[...], xt, preferred_element_type=jnp.float32)
    qrt = jnp.dot(wqr_ref[...], xt, preferred_element_type=jnp.float32)

    col = jax.lax.broadcasted_iota(jnp.int32, (_L, _S), 1)
    masks = [((col >> bb) & 1) == 1 for bb in range(6)]

    aw = []
    for h in range(_H):
        sw = jnp.dot(ks, qwt[h * _DQK:(h + 1) * _DQK, :],
                     preferred_element_type=jnp.float32) * scale
        aw.append(_shear_t(_softmax_cols(sw), masks))

    a = 1.0 - (((aw[0] + aw[1]) + (aw[2] + aw[3]))
               + ((aw[4] + aw[5]) + (aw[6] + aw[7])))

    vmem = jnp.zeros((_L, _DV, _S), jnp.float32)
    for h in range(_H):
        vmem = vmem + aw[h][:, None, :] * vt[h * _DV:(h + 1) * _DV, :][None, :, :]
    v_scr[...] = vmem

    _G = 4
    for lg in range(0, _L, _G):
        vs = v_scr[lg:lg + _G]
        asl = a[lg:lg + _G, :]
        d = 1
        while d < _S:
            v_sh = jnp.concatenate(
                [jnp.zeros((_G, _DV, d), jnp.float32), vs[:, :, :-d]], axis=2)
            vs = vs + asl[:, None, :] * v_sh
            a_sh = jnp.concatenate(
                [jnp.ones((_G, d), jnp.float32), asl[:, :-d]], axis=1)
            asl = asl * a_sh
            d *= 2
        carry = jnp.broadcast_to(y_scr[lg:lg + _G], (_G, _DV, _S))
        vs = vs + asl[:, None, :] * carry
        v_scr[lg:lg + _G] = vs
        y_scr[lg:lg + _G] = vs[:, :, _S - 1:_S]

    ymem = v_scr[...]

    outs = []
    for h in range(_H):
        sr = jnp.dot(ks, qrt[h * _DQK:(h + 1) * _DQK, :],
                     preferred_element_type=jnp.float32) * scale
        ar = _softmax_cols(sr)
        outs.append(jnp.sum(ar[:, None, :] * ymem, axis=0))

    y_cat = jnp.concatenate(outs, axis=0).T
    o_ref[0] = jnp.dot(y_cat, wo_ref[...], preferred_element_type=jnp.float32)


def kernel(x, k_star, w_qw, w_qr, w_v, w_o):
    wqw = w_qw.reshape(_H * _DQK, _C)
    wqr = w_qr.reshape(_H * _DQK, _C)
    wv = w_v.reshape(_H * _DV, _C)

    return pl.pallas_call(
        _body,
        grid=(_N, _NB),
        in_specs=[
            pl.BlockSpec((1, _S, _C), lambda n, b: (n, b, 0)),
            pl.BlockSpec((_H * _DQK, _C), lambda n, b: (0, 0)),
            pl.BlockSpec((_H * _DQK, _C), lambda n, b: (0, 0)),
            pl.BlockSpec((_H * _DV, _C), lambda n, b: (0, 0)),
            pl.BlockSpec((_L, _DQK), lambda n, b: (0, 0)),
            pl.BlockSpec((_H * _DV, _C), lambda n, b: (0, 0)),
        ],
        out_specs=pl.BlockSpec((1, _S, _C), lambda n, b: (n, b, 0)),
        out_shape=jax.ShapeDtypeStruct((_N, _T, _C), jnp.float32),
        scratch_shapes=[
            pltpu.VMEM((_L, _DV, 1), jnp.float32),
            pltpu.VMEM((_L, _DV, _S), jnp.float32),
        ],
        compiler_params=pltpu.CompilerParams(
            dimension_semantics=("parallel", "arbitrary"),
            vmem_limit_bytes=100 * 1024 * 1024,
        ),
    )(x, wqw, wqr, wv, k_star, w_o)
